# Initial kernel scaffold; baseline (speedup 1.0000x reference)
import jax
import jax.numpy as jnp
from jax.experimental import pallas as pl


def kernel(x, nn_param, labels):
    raise NotImplementedError("write your pallas kernel here")


# speedup vs baseline: 1673.6479x; 1673.6479x over previous
import numpy as np
import jax
import jax.numpy as jnp
from jax.experimental import pallas as pl
from jax.experimental.pallas import tpu as pltpu

SR = 44100
T = 88200
N1 = 294
N2 = 300
B = 32
C = 2
LN10 = float(np.log(10.0))


def _dft_consts():
    def dftm(n):
        jk = np.outer(np.arange(n), np.arange(n)) % n
        w = np.exp(-2j * np.pi * jk / n)
        return w.real.astype(np.float32), w.imag.astype(np.float32)

    f1re, f1im = dftm(N1)
    f2re, f2im = dftm(N2)
    jn = np.outer(np.arange(N1), np.arange(N2)) % T
    tw = np.exp(-2j * np.pi * jn / T)
    return (f1re, f1im, f2re, f2im,
            tw.real.astype(np.float32), tw.imag.astype(np.float32))


_F1RE, _F1IM, _F2RE, _F2IM, _TWRE, _TWIM = _dft_consts()

_KLIN = np.arange(T)
_FOLD = np.minimum(_KLIN, T - _KLIN).astype(np.int32)
_SGN_K = (np.where(_KLIN <= T // 2, 1.0, -1.0)
          .astype(np.float32).reshape(N2, N1).T.copy())


def _kmat(v_rfft):
    return jnp.take(v_rfft, _FOLD).reshape(N2, N1).T


def _fx_body(x_ref, p_ref, lab_ref,
             f1re_ref, f1im_ref, f2re_ref, f2im_ref, twre_ref, twim_ref,
             wl_ref, wm_ref, wh_ref, m2pf_ref, sgn_ref, noise_ref, t_ref,
             out_ref):
    b = pl.program_id(0)
    lab = lab_ref[b]

    def mm(a, bb):
        return jax.lax.dot_general(
            a, bb, (((1,), (0,)), ((), ())),
            preferred_element_type=jnp.float32,
            precision=jax.lax.Precision.HIGHEST)

    def fwd(a):
        f1re = f1re_ref[...]
        f1im = f1im_ref[...]
        twre = twre_ref[...]
        twim = twim_ref[...]
        br = mm(f1re, a)
        bi = mm(f1im, a)
        cr = br * twre - bi * twim
        ci = br * twim + bi * twre
        f2re = f2re_ref[...]
        f2im = f2im_ref[...]
        return (mm(cr, f2re) - mm(ci, f2im),
                mm(cr, f2im) + mm(ci, f2re))

    def inv(yr, yi):
        f2re = f2re_ref[...]
        f2im = f2im_ref[...]
        dr = mm(yr, f2re) + mm(yi, f2im)
        di = mm(yi, f2re) - mm(yr, f2im)
        twre = twre_ref[...]
        twim = twim_ref[...]
        er = dr * twre + di * twim
        ei = di * twre - dr * twim
        f1re = f1re_ref[...]
        f1im = f1im_ref[...]
        return (mm(f1re, er) + mm(f1im, ei)) * jnp.float32(1.0 / T)

    def xc(c):
        return x_ref[0, c]

    def br_eq():
        gl = p_ref[b, 0]
        gm = p_ref[b, 1]
        gh = p_ref[b, 2]
        curve = wl_ref[...] * gl + wm_ref[...] * gm + wh_ref[...] * gh
        g = jnp.exp(curve * jnp.float32(LN10 / 20.0))
        for c in range(C):
            xr, xi = fwd(xc(c))
            out_ref[0, c] = inv(xr * g, xi * g)

    def br_dist():
        g = p_ref[b, 3]
        for c in range(C):
            out_ref[0, c] = jnp.tanh(g * xc(c))

    def br_mbc():
        for c in range(C):
            xr, xi = fwd(xc(c))
            acc = jnp.zeros((N1, N2), jnp.float32)
            for i, w_ref in enumerate((wl_ref, wm_ref, wh_ref)):
                w = w_ref[...]
                band = inv(xr * w, xi * w)
                ms = jnp.sum(band * band) * jnp.float32(1.0 / T)
                thr = p_ref[b, 4 + 2 * i]
                ratio = p_ref[b, 5 + 2 * i]
                msm = jnp.full((N1, N2), ms, jnp.float32)
                rms = jnp.sqrt(msm + 1e-8)
                lvl = jnp.log(rms + 1e-8) * jnp.float32(20.0 / LN10)
                gdb = jnp.where(lvl > thr,
                                (thr - lvl) * (1.0 - 1.0 / ratio), 0.0)
                acc = acc + band * jnp.exp(gdb * jnp.float32(LN10 / 20.0))
            out_ref[0, c] = acc

    def br_gain():
        g = p_ref[b, 10]
        for c in range(C):
            out_ref[0, c] = g * xc(c)

    def br_lim():
        thr = p_ref[b, 11]
        for c in range(C):
            out_ref[0, c] = thr * jnp.tanh(xc(c) / thr)

    def br_img():
        w = p_ref[b, 12]
        x0 = xc(0)
        x1 = xc(1)
        mid = 0.5 * (x0 + x1)
        side = 0.5 * (x0 - x1)
        out_ref[0, 0] = mid + w * side
        out_ref[0, 1] = mid - w * side

    def br_delay():
        d = p_ref[b, 13]
        wet = p_ref[b, 14]
        ang = m2pf_ref[...] * d
        hre = jnp.cos(ang)
        him = sgn_ref[...] * jnp.sin(ang)
        for c in range(C):
            x0 = xc(c)
            xr, xi = fwd(x0)
            wet_sig = inv(xr * hre - xi * him, xr * him + xi * hre)
            out_ref[0, c] = (1.0 - wet) * x0 + wet * wet_sig

    def br_rev():
        de = p_ref[b, 15]
        wet = p_ref[b, 16]
        ir = noise_ref[...] * jnp.exp((-t_ref[...]) / de)
        e = jnp.sum(ir * ir)
        den = jnp.sqrt(jnp.full((N1, N2), e, jnp.float32)) + 1e-6
        irn = ir / den
        irr, iri = fwd(irn)
        for c in range(C):
            x0 = xc(c)
            xr, xi = fwd(x0)
            wet_sig = inv(xr * irr - xi * iri, xr * iri + xi * irr)
            out_ref[0, c] = (1.0 - wet) * x0 + wet * wet_sig

    branches = (br_eq, br_dist, br_mbc, br_gain, br_lim, br_img,
                br_delay, br_rev)
    for i, br in enumerate(branches):
        pl.when(lab == i)(br)


def kernel(x, nn_param, labels):
    freqs = jnp.fft.rfftfreq(T, 1.0 / SR)
    lf = jnp.log10(freqs + 1e-3)
    t1 = jax.nn.sigmoid((lf - np.log10(250.0)) * 8.0)
    t2 = jax.nn.sigmoid((lf - np.log10(4000.0)) * 8.0)
    w_low = 1.0 - t1
    w_high = t2
    w_mid = t1 * (1.0 - t2)
    wl_k = _kmat(w_low)
    wm_k = _kmat(w_mid)
    wh_k = _kmat(w_high)
    m2pf_k = _kmat((-2.0 * jnp.pi) * freqs)
    noise_n = jax.random.normal(jax.random.key(42), (T,),
                                dtype=jnp.float32).reshape(N1, N2)
    t_n = (jnp.arange(T, dtype=jnp.float32) / SR).reshape(N1, N2)

    p = nn_param

    def dn(v, lo, hi):
        return lo + v * (hi - lo)

    ptab = jnp.stack([
        dn(p[:, 0], -12.0, 12.0),
        dn(p[:, 1], -12.0, 12.0),
        dn(p[:, 2], -12.0, 12.0),
        10.0 ** (dn(p[:, 3], 0.0, 8.0) / 20.0),
        dn(p[:, 4], -30.0, -5.0),
        dn(p[:, 5], 1.5, 6.0),
        dn(p[:, 6], -30.0, -5.0),
        dn(p[:, 7], 1.5, 6.0),
        dn(p[:, 8], -30.0, -5.0),
        dn(p[:, 9], 1.5, 6.0),
        10.0 ** (dn(p[:, 10], 6.0, 12.0) / 20.0),
        10.0 ** (dn(p[:, 11], -20.0, -1e-3) / 20.0),
        p[:, 12],
        dn(p[:, 13], 0.0, 300.0) / 1000.0,
        dn(p[:, 14], 0.1, 0.7),
        dn(p[:, 15], 0.05, 1.0) + 1e-4,
        dn(p[:, 16], 0.1, 0.7),
    ], axis=1).astype(jnp.float32)

    x4 = x.reshape(B, C, N1, N2)

    def cmat(a):
        nd = a.ndim
        return pl.BlockSpec(a.shape, lambda b, _n=nd: (0,) * _n)

    consts = (jnp.asarray(_F1RE), jnp.asarray(_F1IM),
              jnp.asarray(_F2RE), jnp.asarray(_F2IM),
              jnp.asarray(_TWRE), jnp.asarray(_TWIM),
              wl_k, wm_k, wh_k, m2pf_k, jnp.asarray(_SGN_K),
              noise_n, t_n)

    out4 = pl.pallas_call(
        _fx_body,
        grid=(B,),
        in_specs=[
            pl.BlockSpec((1, C, N1, N2), lambda b: (b, 0, 0, 0)),
            pl.BlockSpec(memory_space=pltpu.SMEM),
            pl.BlockSpec(memory_space=pltpu.SMEM),
        ] + [cmat(a) for a in consts],
        out_specs=pl.BlockSpec((1, C, N1, N2), lambda b: (b, 0, 0, 0)),
        out_shape=jax.ShapeDtypeStruct((B, C, N1, N2), jnp.float32),
    )(x4, ptab, labels, *consts)

    out = out4.reshape(B, C, T)
    activate = jax.nn.one_hot(labels, 8, dtype=x.dtype)
    return (out, nn_param, activate, labels)


# speedup vs baseline: 1679.1317x; 1.0033x over previous
import numpy as np
import jax
import jax.numpy as jnp
from jax.experimental import pallas as pl
from jax.experimental.pallas import tpu as pltpu

SR = 44100
T = 88200
N1 = 294
N2 = 300
B = 32
C = 2
LN10 = float(np.log(10.0))


def _dft_consts():
    def dftm(n):
        jk = np.outer(np.arange(n), np.arange(n)) % n
        w = np.exp(-2j * np.pi * jk / n)
        return w.real.astype(np.float32), w.imag.astype(np.float32)

    f1re, f1im = dftm(N1)
    f2re, f2im = dftm(N2)
    jn = np.outer(np.arange(N1), np.arange(N2)) % T
    tw = np.exp(-2j * np.pi * jn / T)
    return (f1re, f1im, f2re, f2im,
            tw.real.astype(np.float32), tw.imag.astype(np.float32))


_F1RE, _F1IM, _F2RE, _F2IM, _TWRE, _TWIM = _dft_consts()
_TW2RE = np.concatenate([_TWRE, _TWRE], axis=1)
_TW2IM = np.concatenate([_TWIM, _TWIM], axis=1)
_TWVRE = np.concatenate([_TWRE, _TWRE], axis=0)
_TWVIM = np.concatenate([_TWIM, _TWIM], axis=0)

_KLIN = np.arange(T)
_FOLD = np.minimum(_KLIN, T - _KLIN).astype(np.int32)
_SGN_K = (np.where(_KLIN <= T // 2, 1.0, -1.0)
          .astype(np.float32).reshape(N2, N1).T.copy())
_SGN_V = np.concatenate([_SGN_K, _SGN_K], axis=0)


def _kmat_v(v_rfft):
    m = jnp.take(v_rfft, _FOLD).reshape(N2, N1).T
    return jnp.concatenate([m, m], axis=0)


def _fx_body(x_ref, p_ref, lab_ref,
             f1re_ref, f1im_ref, f2re_ref, f2im_ref,
             twre_ref, twim_ref, tw2re_ref, tw2im_ref,
             twvre_ref, twvim_ref,
             wl_ref, wm_ref, wh_ref, m2pf_ref, sgn_ref, noise_ref, t_ref,
             out_ref):
    b = pl.program_id(0)
    lab = lab_ref[b]

    def mm(a, bb):
        return jax.lax.dot_general(
            a, bb, (((1,), (0,)), ((), ())),
            preferred_element_type=jnp.float32,
            precision=jax.lax.Precision.HIGHEST)

    def fwd2():
        a2 = jnp.concatenate([x_ref[0, 0], x_ref[0, 1]], axis=1)
        br = mm(f1re_ref[...], a2)
        bi = mm(f1im_ref[...], a2)
        cr = br * tw2re_ref[...] - bi * tw2im_ref[...]
        ci = br * tw2im_ref[...] + bi * tw2re_ref[...]
        crv = jnp.concatenate([cr[:, :N2], cr[:, N2:]], axis=0)
        civ = jnp.concatenate([ci[:, :N2], ci[:, N2:]], axis=0)
        f2re = f2re_ref[...]
        f2im = f2im_ref[...]
        return (mm(crv, f2re) - mm(civ, f2im),
                mm(crv, f2im) + mm(civ, f2re))

    def inv2(yr, yi):
        f2re = f2re_ref[...]
        f2im = f2im_ref[...]
        dr = mm(yr, f2re) + mm(yi, f2im)
        di = mm(yi, f2re) - mm(yr, f2im)
        er = dr * twvre_ref[...] + di * twvim_ref[...]
        ei = di * twvre_ref[...] - dr * twvim_ref[...]
        erh = jnp.concatenate([er[:N1], er[N1:]], axis=1)
        eih = jnp.concatenate([ei[:N1], ei[N1:]], axis=1)
        out = mm(f1re_ref[...], erh) + mm(f1im_ref[...], eih)
        return out * jnp.float32(1.0 / T)

    def fwd1(a):
        br = mm(f1re_ref[...], a)
        bi = mm(f1im_ref[...], a)
        cr = br * twre_ref[...] - bi * twim_ref[...]
        ci = br * twim_ref[...] + bi * twre_ref[...]
        f2re = f2re_ref[...]
        f2im = f2im_ref[...]
        return (mm(cr, f2re) - mm(ci, f2im),
                mm(cr, f2im) + mm(ci, f2re))

    def store2(res):
        out_ref[0, 0] = res[:, :N2]
        out_ref[0, 1] = res[:, N2:]

    def br_eq():
        gl = p_ref[b, 0]
        gm = p_ref[b, 1]
        gh = p_ref[b, 2]
        curve = (wl_ref[...] * gl + wm_ref[...] * gm + wh_ref[...] * gh)
        g = jnp.exp(curve * jnp.float32(LN10 / 20.0))
        xr, xi = fwd2()
        store2(inv2(xr * g, xi * g))

    def br_dist():
        g = p_ref[b, 3]
        for c in range(C):
            out_ref[0, c] = jnp.tanh(g * x_ref[0, c])

    def br_mbc():
        xr, xi = fwd2()
        acc = jnp.zeros((N1, 2 * N2), jnp.float32)
        for i, w_ref in enumerate((wl_ref, wm_ref, wh_ref)):
            w = w_ref[...]
            band = inv2(xr * w, xi * w)
            thr = p_ref[b, 4 + 2 * i]
            ratio = p_ref[b, 5 + 2 * i]
            gs = []
            for c in range(C):
                bc = band[:, c * N2:(c + 1) * N2]
                ms = jnp.sum(bc * bc) * jnp.float32(1.0 / T)
                msm = jnp.full((N1, N2), ms, jnp.float32)
                rms = jnp.sqrt(msm + 1e-8)
                lvl = jnp.log(rms + 1e-8) * jnp.float32(20.0 / LN10)
                gdb = jnp.where(lvl > thr,
                                (thr - lvl) * (1.0 - 1.0 / ratio), 0.0)
                gs.append(jnp.exp(gdb * jnp.float32(LN10 / 20.0)))
            acc = acc + band * jnp.concatenate(gs, axis=1)
        store2(acc)

    def br_gain():
        g = p_ref[b, 10]
        for c in range(C):
            out_ref[0, c] = g * x_ref[0, c]

    def br_lim():
        thr = p_ref[b, 11]
        for c in range(C):
            out_ref[0, c] = thr * jnp.tanh(x_ref[0, c] / thr)

    def br_img():
        w = p_ref[b, 12]
        x0 = x_ref[0, 0]
        x1 = x_ref[0, 1]
        mid = 0.5 * (x0 + x1)
        side = 0.5 * (x0 - x1)
        out_ref[0, 0] = mid + w * side
        out_ref[0, 1] = mid - w * side

    def br_delay():
        d = p_ref[b, 13]
        wet = p_ref[b, 14]
        ang = m2pf_ref[...] * d
        hre = jnp.cos(ang)
        him = sgn_ref[...] * jnp.sin(ang)
        xr, xi = fwd2()
        wet_sig = inv2(xr * hre - xi * him, xr * him + xi * hre)
        x2 = jnp.concatenate([x_ref[0, 0], x_ref[0, 1]], axis=1)
        store2((1.0 - wet) * x2 + wet * wet_sig)

    def br_rev():
        de = p_ref[b, 15]
        wet = p_ref[b, 16]
        ir = noise_ref[...] * jnp.exp((-t_ref[...]) / de)
        e = jnp.sum(ir * ir)
        den = jnp.sqrt(jnp.full((N1, N2), e, jnp.float32)) + 1e-6
        irr, iri = fwd1(ir / den)
        irr2 = jnp.concatenate([irr, irr], axis=0)
        iri2 = jnp.concatenate([iri, iri], axis=0)
        xr, xi = fwd2()
        wet_sig = inv2(xr * irr2 - xi * iri2, xr * iri2 + xi * irr2)
        x2 = jnp.concatenate([x_ref[0, 0], x_ref[0, 1]], axis=1)
        store2((1.0 - wet) * x2 + wet * wet_sig)

    branches = (br_eq, br_dist, br_mbc, br_gain, br_lim, br_img,
                br_delay, br_rev)
    for i, br in enumerate(branches):
        pl.when(lab == i)(br)


def kernel(x, nn_param, labels):
    freqs = jnp.fft.rfftfreq(T, 1.0 / SR)
    lf = jnp.log10(freqs + 1e-3)
    t1 = jax.nn.sigmoid((lf - np.log10(250.0)) * 8.0)
    t2 = jax.nn.sigmoid((lf - np.log10(4000.0)) * 8.0)
    w_low = 1.0 - t1
    w_high = t2
    w_mid = t1 * (1.0 - t2)
    wl_v = _kmat_v(w_low)
    wm_v = _kmat_v(w_mid)
    wh_v = _kmat_v(w_high)
    m2pf_v = _kmat_v((-2.0 * jnp.pi) * freqs)
    noise_n = jax.random.normal(jax.random.key(42), (T,),
                                dtype=jnp.float32).reshape(N1, N2)
    t_n = (jnp.arange(T, dtype=jnp.float32) / SR).reshape(N1, N2)

    p = nn_param

    def dn(v, lo, hi):
        return lo + v * (hi - lo)

    ptab = jnp.stack([
        dn(p[:, 0], -12.0, 12.0),
        dn(p[:, 1], -12.0, 12.0),
        dn(p[:, 2], -12.0, 12.0),
        10.0 ** (dn(p[:, 3], 0.0, 8.0) / 20.0),
        dn(p[:, 4], -30.0, -5.0),
        dn(p[:, 5], 1.5, 6.0),
        dn(p[:, 6], -30.0, -5.0),
        dn(p[:, 7], 1.5, 6.0),
        dn(p[:, 8], -30.0, -5.0),
        dn(p[:, 9], 1.5, 6.0),
        10.0 ** (dn(p[:, 10], 6.0, 12.0) / 20.0),
        10.0 ** (dn(p[:, 11], -20.0, -1e-3) / 20.0),
        p[:, 12],
        dn(p[:, 13], 0.0, 300.0) / 1000.0,
        dn(p[:, 14], 0.1, 0.7),
        dn(p[:, 15], 0.05, 1.0) + 1e-4,
        dn(p[:, 16], 0.1, 0.7),
    ], axis=1).astype(jnp.float32)

    x4 = x.reshape(B, C, N1, N2)

    def cmat(a):
        nd = a.ndim
        return pl.BlockSpec(a.shape, lambda b, _n=nd: (0,) * _n)

    consts = (jnp.asarray(_F1RE), jnp.asarray(_F1IM),
              jnp.asarray(_F2RE), jnp.asarray(_F2IM),
              jnp.asarray(_TWRE), jnp.asarray(_TWIM),
              jnp.asarray(_TW2RE), jnp.asarray(_TW2IM),
              jnp.asarray(_TWVRE), jnp.asarray(_TWVIM),
              wl_v, wm_v, wh_v, m2pf_v, jnp.asarray(_SGN_V),
              noise_n, t_n)

    out4 = pl.pallas_call(
        _fx_body,
        grid=(B,),
        in_specs=[
            pl.BlockSpec((1, C, N1, N2), lambda b: (b, 0, 0, 0)),
            pl.BlockSpec(memory_space=pltpu.SMEM),
            pl.BlockSpec(memory_space=pltpu.SMEM),
        ] + [cmat(a) for a in consts],
        out_specs=pl.BlockSpec((1, C, N1, N2), lambda b: (b, 0, 0, 0)),
        out_shape=jax.ShapeDtypeStruct((B, C, N1, N2), jnp.float32),
    )(x4, ptab, labels, *consts)

    out = out4.reshape(B, C, T)
    activate = jax.nn.one_hot(labels, 8, dtype=x.dtype)
    return (out, nn_param, activate, labels)


# speedup vs baseline: 5738.8356x; 3.4177x over previous
import numpy as np
import jax
import jax.numpy as jnp
from jax.experimental import pallas as pl
from jax.experimental.pallas import tpu as pltpu

SR = 44100
T = 88200
N1 = 294
N2 = 300
B = 32
C = 2
LN10 = float(np.log(10.0))


def _dft_consts():
    def dftm(n):
        jk = np.outer(np.arange(n), np.arange(n)) % n
        w = np.exp(-2j * np.pi * jk / n)
        return w.real.astype(np.float32), w.imag.astype(np.float32)

    f1re, f1im = dftm(N1)
    f2re, f2im = dftm(N2)
    jn = np.outer(np.arange(N1), np.arange(N2)) % T
    tw = np.exp(-2j * np.pi * jn / T)
    return (f1re, f1im, f2re, f2im,
            tw.real.astype(np.float32), tw.imag.astype(np.float32))


_F1RE, _F1IM, _F2RE, _F2IM, _TWRE, _TWIM = _dft_consts()
_TW2RE = np.concatenate([_TWRE, _TWRE], axis=1)
_TW2IM = np.concatenate([_TWIM, _TWIM], axis=1)
_TWVRE = np.concatenate([_TWRE, _TWRE], axis=0)
_TWVIM = np.concatenate([_TWIM, _TWIM], axis=0)

_KLIN = np.arange(T)
_FOLD = np.minimum(_KLIN, T - _KLIN).astype(np.int32)
_SGN_K = (np.where(_KLIN <= T // 2, 1.0, -1.0)
          .astype(np.float32).reshape(N2, N1).T.copy())
_SGN_V = np.concatenate([_SGN_K, _SGN_K], axis=0)


def _kmat_v(v_rfft):
    m = jnp.take(v_rfft, _FOLD).reshape(N2, N1).T
    return jnp.concatenate([m, m], axis=0)


def _fx_body(x_ref, p_ref, lab_ref,
             f1re_ref, f1im_ref, f2re_ref, f2im_ref,
             twre_ref, twim_ref, tw2re_ref, tw2im_ref,
             twvre_ref, twvim_ref,
             wl_ref, wm_ref, wh_ref, m2pf_ref, sgn_ref, noise_ref, t_ref,
             out_ref):
    b = pl.program_id(0)
    lab = lab_ref[b]

    def mm(a, bb):
        return jax.lax.dot_general(
            a, bb, (((1,), (0,)), ((), ())),
            preferred_element_type=jnp.float32,
            precision=jax.lax.Precision.DEFAULT)

    def fwd2():
        a2 = jnp.concatenate([x_ref[0, 0], x_ref[0, 1]], axis=1)
        br = mm(f1re_ref[...], a2)
        bi = mm(f1im_ref[...], a2)
        cr = br * tw2re_ref[...] - bi * tw2im_ref[...]
        ci = br * tw2im_ref[...] + bi * tw2re_ref[...]
        crv = jnp.concatenate([cr[:, :N2], cr[:, N2:]], axis=0)
        civ = jnp.concatenate([ci[:, :N2], ci[:, N2:]], axis=0)
        f2re = f2re_ref[...]
        f2im = f2im_ref[...]
        return (mm(crv, f2re) - mm(civ, f2im),
                mm(crv, f2im) + mm(civ, f2re))

    def inv2(yr, yi):
        f2re = f2re_ref[...]
        f2im = f2im_ref[...]
        dr = mm(yr, f2re) + mm(yi, f2im)
        di = mm(yi, f2re) - mm(yr, f2im)
        er = dr * twvre_ref[...] + di * twvim_ref[...]
        ei = di * twvre_ref[...] - dr * twvim_ref[...]
        erh = jnp.concatenate([er[:N1], er[N1:]], axis=1)
        eih = jnp.concatenate([ei[:N1], ei[N1:]], axis=1)
        out = mm(f1re_ref[...], erh) + mm(f1im_ref[...], eih)
        return out * jnp.float32(1.0 / T)

    def fwd1(a):
        br = mm(f1re_ref[...], a)
        bi = mm(f1im_ref[...], a)
        cr = br * twre_ref[...] - bi * twim_ref[...]
        ci = br * twim_ref[...] + bi * twre_ref[...]
        f2re = f2re_ref[...]
        f2im = f2im_ref[...]
        return (mm(cr, f2re) - mm(ci, f2im),
                mm(cr, f2im) + mm(ci, f2re))

    def store2(res):
        out_ref[0, 0] = res[:, :N2]
        out_ref[0, 1] = res[:, N2:]

    def br_eq():
        gl = p_ref[b, 0]
        gm = p_ref[b, 1]
        gh = p_ref[b, 2]
        curve = (wl_ref[...] * gl + wm_ref[...] * gm + wh_ref[...] * gh)
        g = jnp.exp(curve * jnp.float32(LN10 / 20.0))
        xr, xi = fwd2()
        store2(inv2(xr * g, xi * g))

    def br_dist():
        g = p_ref[b, 3]
        for c in range(C):
            out_ref[0, c] = jnp.tanh(g * x_ref[0, c])

    def br_mbc():
        xr, xi = fwd2()
        acc = jnp.zeros((N1, 2 * N2), jnp.float32)
        for i, w_ref in enumerate((wl_ref, wm_ref, wh_ref)):
            w = w_ref[...]
            band = inv2(xr * w, xi * w)
            thr = p_ref[b, 4 + 2 * i]
            ratio = p_ref[b, 5 + 2 * i]
            gs = []
            for c in range(C):
                bc = band[:, c * N2:(c + 1) * N2]
                ms = jnp.sum(bc * bc) * jnp.float32(1.0 / T)
                msm = jnp.full((N1, N2), ms, jnp.float32)
                rms = jnp.sqrt(msm + 1e-8)
                lvl = jnp.log(rms + 1e-8) * jnp.float32(20.0 / LN10)
                gdb = jnp.where(lvl > thr,
                                (thr - lvl) * (1.0 - 1.0 / ratio), 0.0)
                gs.append(jnp.exp(gdb * jnp.float32(LN10 / 20.0)))
            acc = acc + band * jnp.concatenate(gs, axis=1)
        store2(acc)

    def br_gain():
        g = p_ref[b, 10]
        for c in range(C):
            out_ref[0, c] = g * x_ref[0, c]

    def br_lim():
        thr = p_ref[b, 11]
        for c in range(C):
            out_ref[0, c] = thr * jnp.tanh(x_ref[0, c] / thr)

    def br_img():
        w = p_ref[b, 12]
        x0 = x_ref[0, 0]
        x1 = x_ref[0, 1]
        mid = 0.5 * (x0 + x1)
        side = 0.5 * (x0 - x1)
        out_ref[0, 0] = mid + w * side
        out_ref[0, 1] = mid - w * side

    def br_delay():
        d = p_ref[b, 13]
        wet = p_ref[b, 14]
        ang = m2pf_ref[...] * d
        hre = jnp.cos(ang)
        him = sgn_ref[...] * jnp.sin(ang)
        xr, xi = fwd2()
        wet_sig = inv2(xr * hre - xi * him, xr * him + xi * hre)
        x2 = jnp.concatenate([x_ref[0, 0], x_ref[0, 1]], axis=1)
        store2((1.0 - wet) * x2 + wet * wet_sig)

    def br_rev():
        de = p_ref[b, 15]
        wet = p_ref[b, 16]
        ir = noise_ref[...] * jnp.exp((-t_ref[...]) / de)
        e = jnp.sum(ir * ir)
        den = jnp.sqrt(jnp.full((N1, N2), e, jnp.float32)) + 1e-6
        irr, iri = fwd1(ir / den)
        irr2 = jnp.concatenate([irr, irr], axis=0)
        iri2 = jnp.concatenate([iri, iri], axis=0)
        xr, xi = fwd2()
        wet_sig = inv2(xr * irr2 - xi * iri2, xr * iri2 + xi * irr2)
        x2 = jnp.concatenate([x_ref[0, 0], x_ref[0, 1]], axis=1)
        store2((1.0 - wet) * x2 + wet * wet_sig)

    branches = (br_eq, br_dist, br_mbc, br_gain, br_lim, br_img,
                br_delay, br_rev)
    for i, br in enumerate(branches):
        pl.when(lab == i)(br)


def kernel(x, nn_param, labels):
    freqs = jnp.fft.rfftfreq(T, 1.0 / SR)
    lf = jnp.log10(freqs + 1e-3)
    t1 = jax.nn.sigmoid((lf - np.log10(250.0)) * 8.0)
    t2 = jax.nn.sigmoid((lf - np.log10(4000.0)) * 8.0)
    w_low = 1.0 - t1
    w_high = t2
    w_mid = t1 * (1.0 - t2)
    wl_v = _kmat_v(w_low)
    wm_v = _kmat_v(w_mid)
    wh_v = _kmat_v(w_high)
    m2pf_v = _kmat_v((-2.0 * jnp.pi) * freqs)
    noise_n = jax.random.normal(jax.random.key(42), (T,),
                                dtype=jnp.float32).reshape(N1, N2)
    t_n = (jnp.arange(T, dtype=jnp.float32) / SR).reshape(N1, N2)

    p = nn_param

    def dn(v, lo, hi):
        return lo + v * (hi - lo)

    ptab = jnp.stack([
        dn(p[:, 0], -12.0, 12.0),
        dn(p[:, 1], -12.0, 12.0),
        dn(p[:, 2], -12.0, 12.0),
        10.0 ** (dn(p[:, 3], 0.0, 8.0) / 20.0),
        dn(p[:, 4], -30.0, -5.0),
        dn(p[:, 5], 1.5, 6.0),
        dn(p[:, 6], -30.0, -5.0),
        dn(p[:, 7], 1.5, 6.0),
        dn(p[:, 8], -30.0, -5.0),
        dn(p[:, 9], 1.5, 6.0),
        10.0 ** (dn(p[:, 10], 6.0, 12.0) / 20.0),
        10.0 ** (dn(p[:, 11], -20.0, -1e-3) / 20.0),
        p[:, 12],
        dn(p[:, 13], 0.0, 300.0) / 1000.0,
        dn(p[:, 14], 0.1, 0.7),
        dn(p[:, 15], 0.05, 1.0) + 1e-4,
        dn(p[:, 16], 0.1, 0.7),
    ], axis=1).astype(jnp.float32)

    x4 = x.reshape(B, C, N1, N2)

    def cmat(a):
        nd = a.ndim
        return pl.BlockSpec(a.shape, lambda b, _n=nd: (0,) * _n)

    consts = (jnp.asarray(_F1RE), jnp.asarray(_F1IM),
              jnp.asarray(_F2RE), jnp.asarray(_F2IM),
              jnp.asarray(_TWRE), jnp.asarray(_TWIM),
              jnp.asarray(_TW2RE), jnp.asarray(_TW2IM),
              jnp.asarray(_TWVRE), jnp.asarray(_TWVIM),
              wl_v, wm_v, wh_v, m2pf_v, jnp.asarray(_SGN_V),
              noise_n, t_n)

    out4 = pl.pallas_call(
        _fx_body,
        grid=(B,),
        in_specs=[
            pl.BlockSpec((1, C, N1, N2), lambda b: (b, 0, 0, 0)),
            pl.BlockSpec(memory_space=pltpu.SMEM),
            pl.BlockSpec(memory_space=pltpu.SMEM),
        ] + [cmat(a) for a in consts],
        out_specs=pl.BlockSpec((1, C, N1, N2), lambda b: (b, 0, 0, 0)),
        out_shape=jax.ShapeDtypeStruct((B, C, N1, N2), jnp.float32),
    )(x4, ptab, labels, *consts)

    out = out4.reshape(B, C, T)
    activate = jax.nn.one_hot(labels, 8, dtype=x.dtype)
    return (out, nn_param, activate, labels)


# speedup vs baseline: 5852.0618x; 1.0197x over previous
import numpy as np
import jax
import jax.numpy as jnp
from jax.experimental import pallas as pl
from jax.experimental.pallas import tpu as pltpu

SR = 44100
T = 88200
N1 = 294
N2 = 300
B = 32
C = 2
LN10 = float(np.log(10.0))


def _dft_consts():
    def dftm(n):
        jk = np.outer(np.arange(n), np.arange(n)) % n
        w = np.exp(-2j * np.pi * jk / n)
        return w.real.astype(np.float32), w.imag.astype(np.float32)

    f1re, f1im = dftm(N1)
    f2re, f2im = dftm(N2)
    jn = np.outer(np.arange(N1), np.arange(N2)) % T
    tw = np.exp(-2j * np.pi * jn / T)
    return (f1re, f1im, f2re, f2im,
            tw.real.astype(np.float32), tw.imag.astype(np.float32))


_F1RE, _F1IM, _F2RE, _F2IM, _TWRE, _TWIM = _dft_consts()
_TW2RE = np.concatenate([_TWRE, _TWRE], axis=1)
_TW2IM = np.concatenate([_TWIM, _TWIM], axis=1)
_TWVRE = np.concatenate([_TWRE, _TWRE], axis=0)
_TWVIM = np.concatenate([_TWIM, _TWIM], axis=0)

_KLIN = np.arange(T)
_FOLD = np.minimum(_KLIN, T - _KLIN).astype(np.int32)
_SGN_K = (np.where(_KLIN <= T // 2, 1.0, -1.0)
          .astype(np.float32).reshape(N2, N1).T.copy())
_SGN_V = np.concatenate([_SGN_K, _SGN_K], axis=0)


def _kmat_v(v_rfft):
    m = jnp.take(v_rfft, _FOLD).reshape(N2, N1).T
    return jnp.concatenate([m, m], axis=0)


def _fx_body(x_ref, p_ref, lab_ref,
             f1re_ref, f1im_ref, f2re_ref, f2im_ref,
             twre_ref, twim_ref, tw2re_ref, tw2im_ref,
             twvre_ref, twvim_ref,
             wl_ref, wm_ref, wh_ref, m2pf_ref, sgn_ref, noise_ref, t_ref,
             out_ref):
    b = pl.program_id(0)
    lab = lab_ref[b]

    def mm(a, bb):
        return jax.lax.dot_general(
            a, bb, (((1,), (0,)), ((), ())),
            preferred_element_type=jnp.float32,
            precision=jax.lax.Precision.DEFAULT)

    def fwd2():
        a2 = jnp.concatenate([x_ref[0, 0], x_ref[0, 1]], axis=1)
        br = mm(f1re_ref[...], a2)
        bi = mm(f1im_ref[...], a2)
        cr = br * tw2re_ref[...] - bi * tw2im_ref[...]
        ci = br * tw2im_ref[...] + bi * tw2re_ref[...]
        crv = jnp.concatenate([cr[:, :N2], cr[:, N2:]], axis=0)
        civ = jnp.concatenate([ci[:, :N2], ci[:, N2:]], axis=0)
        f2re = f2re_ref[...]
        f2im = f2im_ref[...]
        return (mm(crv, f2re) - mm(civ, f2im),
                mm(crv, f2im) + mm(civ, f2re))

    def inv2(yr, yi):
        f2re = f2re_ref[...]
        f2im = f2im_ref[...]
        dr = mm(yr, f2re) + mm(yi, f2im)
        di = mm(yi, f2re) - mm(yr, f2im)
        er = dr * twvre_ref[...] + di * twvim_ref[...]
        ei = di * twvre_ref[...] - dr * twvim_ref[...]
        erh = jnp.concatenate([er[:N1], er[N1:]], axis=1)
        eih = jnp.concatenate([ei[:N1], ei[N1:]], axis=1)
        out = mm(f1re_ref[...], erh) + mm(f1im_ref[...], eih)
        return out * jnp.float32(1.0 / T)

    def fwd1(a):
        br = mm(f1re_ref[...], a)
        bi = mm(f1im_ref[...], a)
        cr = br * twre_ref[...] - bi * twim_ref[...]
        ci = br * twim_ref[...] + bi * twre_ref[...]
        f2re = f2re_ref[...]
        f2im = f2im_ref[...]
        return (mm(cr, f2re) - mm(ci, f2im),
                mm(cr, f2im) + mm(ci, f2re))

    def store2(res):
        out_ref[0, 0] = res[:, :N2]
        out_ref[0, 1] = res[:, N2:]

    def br_eq():
        gl = p_ref[b, 0]
        gm = p_ref[b, 1]
        gh = p_ref[b, 2]
        curve = (wl_ref[...] * gl + wm_ref[...] * gm + wh_ref[...] * gh)
        g = jnp.exp(curve * jnp.float32(LN10 / 20.0))
        xr, xi = fwd2()
        store2(inv2(xr * g, xi * g))

    def br_dist():
        g = p_ref[b, 3]
        for c in range(C):
            out_ref[0, c] = jnp.tanh(g * x_ref[0, c])

    def br_mbc():
        xr, xi = fwd2()
        accr = jnp.zeros((2 * N1, N2), jnp.float32)
        acci = jnp.zeros((2 * N1, N2), jnp.float32)
        for i, w_ref in enumerate((wl_ref, wm_ref, wh_ref)):
            w = w_ref[...]
            sr = xr * w
            si = xi * w
            thr = p_ref[b, 4 + 2 * i]
            ratio = p_ref[b, 5 + 2 * i]
            gs = []
            for c in range(C):
                pr = sr[c * N1:(c + 1) * N1]
                pi = si[c * N1:(c + 1) * N1]
                ms = jnp.sum(pr * pr + pi * pi) * jnp.float32(1.0 / T / T)
                msm = jnp.full((N1, N2), ms, jnp.float32)
                rms = jnp.sqrt(msm + 1e-8)
                lvl = jnp.log(rms + 1e-8) * jnp.float32(20.0 / LN10)
                gdb = jnp.where(lvl > thr,
                                (thr - lvl) * (1.0 - 1.0 / ratio), 0.0)
                gs.append(jnp.exp(gdb * jnp.float32(LN10 / 20.0)))
            g2 = jnp.concatenate(gs, axis=0)
            accr = accr + sr * g2
            acci = acci + si * g2
        store2(inv2(accr, acci))

    def br_gain():
        g = p_ref[b, 10]
        for c in range(C):
            out_ref[0, c] = g * x_ref[0, c]

    def br_lim():
        thr = p_ref[b, 11]
        for c in range(C):
            out_ref[0, c] = thr * jnp.tanh(x_ref[0, c] / thr)

    def br_img():
        w = p_ref[b, 12]
        x0 = x_ref[0, 0]
        x1 = x_ref[0, 1]
        mid = 0.5 * (x0 + x1)
        side = 0.5 * (x0 - x1)
        out_ref[0, 0] = mid + w * side
        out_ref[0, 1] = mid - w * side

    def br_delay():
        d = p_ref[b, 13]
        wet = p_ref[b, 14]
        ang = m2pf_ref[...] * d
        hre = jnp.cos(ang)
        him = sgn_ref[...] * jnp.sin(ang)
        xr, xi = fwd2()
        wet_sig = inv2(xr * hre - xi * him, xr * him + xi * hre)
        x2 = jnp.concatenate([x_ref[0, 0], x_ref[0, 1]], axis=1)
        store2((1.0 - wet) * x2 + wet * wet_sig)

    def br_rev():
        de = p_ref[b, 15]
        wet = p_ref[b, 16]
        ir = noise_ref[...] * jnp.exp((-t_ref[...]) / de)
        e = jnp.sum(ir * ir)
        den = jnp.sqrt(jnp.full((N1, N2), e, jnp.float32)) + 1e-6
        irr, iri = fwd1(ir / den)
        irr2 = jnp.concatenate([irr, irr], axis=0)
        iri2 = jnp.concatenate([iri, iri], axis=0)
        xr, xi = fwd2()
        wet_sig = inv2(xr * irr2 - xi * iri2, xr * iri2 + xi * irr2)
        x2 = jnp.concatenate([x_ref[0, 0], x_ref[0, 1]], axis=1)
        store2((1.0 - wet) * x2 + wet * wet_sig)

    branches = (br_eq, br_dist, br_mbc, br_gain, br_lim, br_img,
                br_delay, br_rev)
    for i, br in enumerate(branches):
        pl.when(lab == i)(br)


def kernel(x, nn_param, labels):
    freqs = jnp.fft.rfftfreq(T, 1.0 / SR)
    lf = jnp.log10(freqs + 1e-3)
    t1 = jax.nn.sigmoid((lf - np.log10(250.0)) * 8.0)
    t2 = jax.nn.sigmoid((lf - np.log10(4000.0)) * 8.0)
    w_low = 1.0 - t1
    w_high = t2
    w_mid = t1 * (1.0 - t2)
    wl_v = _kmat_v(w_low)
    wm_v = _kmat_v(w_mid)
    wh_v = _kmat_v(w_high)
    m2pf_v = _kmat_v((-2.0 * jnp.pi) * freqs)
    noise_n = jax.random.normal(jax.random.key(42), (T,),
                                dtype=jnp.float32).reshape(N1, N2)
    t_n = (jnp.arange(T, dtype=jnp.float32) / SR).reshape(N1, N2)

    p = nn_param

    def dn(v, lo, hi):
        return lo + v * (hi - lo)

    ptab = jnp.stack([
        dn(p[:, 0], -12.0, 12.0),
        dn(p[:, 1], -12.0, 12.0),
        dn(p[:, 2], -12.0, 12.0),
        10.0 ** (dn(p[:, 3], 0.0, 8.0) / 20.0),
        dn(p[:, 4], -30.0, -5.0),
        dn(p[:, 5], 1.5, 6.0),
        dn(p[:, 6], -30.0, -5.0),
        dn(p[:, 7], 1.5, 6.0),
        dn(p[:, 8], -30.0, -5.0),
        dn(p[:, 9], 1.5, 6.0),
        10.0 ** (dn(p[:, 10], 6.0, 12.0) / 20.0),
        10.0 ** (dn(p[:, 11], -20.0, -1e-3) / 20.0),
        p[:, 12],
        dn(p[:, 13], 0.0, 300.0) / 1000.0,
        dn(p[:, 14], 0.1, 0.7),
        dn(p[:, 15], 0.05, 1.0) + 1e-4,
        dn(p[:, 16], 0.1, 0.7),
    ], axis=1).astype(jnp.float32)

    x4 = x.reshape(B, C, N1, N2)

    def cmat(a):
        nd = a.ndim
        return pl.BlockSpec(a.shape, lambda b, _n=nd: (0,) * _n)

    consts = (jnp.asarray(_F1RE), jnp.asarray(_F1IM),
              jnp.asarray(_F2RE), jnp.asarray(_F2IM),
              jnp.asarray(_TWRE), jnp.asarray(_TWIM),
              jnp.asarray(_TW2RE), jnp.asarray(_TW2IM),
              jnp.asarray(_TWVRE), jnp.asarray(_TWVIM),
              wl_v, wm_v, wh_v, m2pf_v, jnp.asarray(_SGN_V),
              noise_n, t_n)

    out4 = pl.pallas_call(
        _fx_body,
        grid=(B,),
        in_specs=[
            pl.BlockSpec((1, C, N1, N2), lambda b: (b, 0, 0, 0)),
            pl.BlockSpec(memory_space=pltpu.SMEM),
            pl.BlockSpec(memory_space=pltpu.SMEM),
        ] + [cmat(a) for a in consts],
        out_specs=pl.BlockSpec((1, C, N1, N2), lambda b: (b, 0, 0, 0)),
        out_shape=jax.ShapeDtypeStruct((B, C, N1, N2), jnp.float32),
    )(x4, ptab, labels, *consts)

    out = out4.reshape(B, C, T)
    activate = jax.nn.one_hot(labels, 8, dtype=x.dtype)
    return (out, nn_param, activate, labels)
